# baseline (speedup 1.0000x reference)
import jax
import jax.numpy as jnp
from jax.experimental import pallas as pl
from jax.experimental.pallas import tpu as pltpu

N_TOKENS = 16384
EMBED_DIM = 2048
LOW_RANK_DIM = 16
OUT_DIM = 64

CHUNK_T = 512
N_BUF = 8
N_CHUNK = N_TOKENS // CHUNK_T


def _body(x_hbm, w1t_ref, w2t_ref, out_ref, xbuf, sems):
    def start(c):
        pltpu.make_async_copy(
            x_hbm.at[pl.ds(c * CHUNK_T, CHUNK_T), :],
            xbuf.at[c % N_BUF],
            sems.at[c % N_BUF],
        ).start()

    def wait(c):
        pltpu.make_async_copy(
            x_hbm.at[pl.ds(c * CHUNK_T, CHUNK_T), :],
            xbuf.at[c % N_BUF],
            sems.at[c % N_BUF],
        ).wait()

    for c in range(N_BUF):
        start(c)
    w1t = w1t_ref[...]
    w2t = w2t_ref[...]
    for c in range(N_CHUNK):
        wait(c)
        h = jnp.dot(xbuf[c % N_BUF], w1t, preferred_element_type=jnp.float32)
        out_ref[c * CHUNK_T:(c + 1) * CHUNK_T, :] = jnp.dot(
            h, w2t, preferred_element_type=jnp.float32)
        if c + N_BUF < N_CHUNK:
            start(c + N_BUF)


def kernel(x, w1, w2):
    n = x.shape[0]
    w1t = w1.T
    w2t = w2.T
    return pl.pallas_call(
        _body,
        in_specs=[
            pl.BlockSpec(memory_space=pl.ANY),
            pl.BlockSpec(memory_space=pltpu.MemorySpace.VMEM),
            pl.BlockSpec(memory_space=pltpu.MemorySpace.VMEM),
        ],
        out_specs=pl.BlockSpec(memory_space=pltpu.MemorySpace.VMEM),
        out_shape=jax.ShapeDtypeStruct((n, OUT_DIM), jnp.float32),
        scratch_shapes=[
            pltpu.VMEM((N_BUF, CHUNK_T, EMBED_DIM), jnp.float32),
            pltpu.SemaphoreType.DMA((N_BUF,)),
        ],
    )(x, w1t, w2t)


# speedup vs baseline: 1.0080x; 1.0080x over previous
import jax
import jax.numpy as jnp
from jax.experimental import pallas as pl
from jax.experimental.pallas import tpu as pltpu

N_TOKENS = 16384
EMBED_DIM = 2048
LOW_RANK_DIM = 16
OUT_DIM = 64

CHUNK_T = 2048
N_BUF = 3
N_CHUNK = N_TOKENS // CHUNK_T


def _body(x_hbm, w1t_ref, w2t_ref, out_hbm, xbuf, obuf, isems, osems):
    def start_in(c):
        pltpu.make_async_copy(
            x_hbm.at[pl.ds(c * CHUNK_T, CHUNK_T), :],
            xbuf.at[c % N_BUF],
            isems.at[c % N_BUF],
        ).start()

    def wait_in(c):
        pltpu.make_async_copy(
            x_hbm.at[pl.ds(c * CHUNK_T, CHUNK_T), :],
            xbuf.at[c % N_BUF],
            isems.at[c % N_BUF],
        ).wait()

    def out_copy(c):
        return pltpu.make_async_copy(
            obuf.at[c % 2],
            out_hbm.at[pl.ds(c * CHUNK_T, CHUNK_T), :],
            osems.at[c % 2],
        )

    for c in range(N_BUF):
        start_in(c)
    w1t = w1t_ref[...]
    w2t = w2t_ref[...]
    for c in range(N_CHUNK):
        wait_in(c)
        h = jnp.dot(xbuf[c % N_BUF], w1t, preferred_element_type=jnp.float32)
        if c >= 2:
            out_copy(c - 2).wait()
        obuf[c % 2] = jnp.dot(h, w2t, preferred_element_type=jnp.float32)
        out_copy(c).start()
        if c + N_BUF < N_CHUNK:
            start_in(c + N_BUF)
    for c in (N_CHUNK - 2, N_CHUNK - 1):
        out_copy(c).wait()


def kernel(x, w1, w2):
    n = x.shape[0]
    w1t = w1.T
    w2t = w2.T
    return pl.pallas_call(
        _body,
        in_specs=[
            pl.BlockSpec(memory_space=pl.ANY),
            pl.BlockSpec(memory_space=pltpu.MemorySpace.VMEM),
            pl.BlockSpec(memory_space=pltpu.MemorySpace.VMEM),
        ],
        out_specs=pl.BlockSpec(memory_space=pl.ANY),
        out_shape=jax.ShapeDtypeStruct((n, OUT_DIM), jnp.float32),
        scratch_shapes=[
            pltpu.VMEM((N_BUF, CHUNK_T, EMBED_DIM), jnp.float32),
            pltpu.VMEM((2, CHUNK_T, OUT_DIM), jnp.float32),
            pltpu.SemaphoreType.DMA((N_BUF,)),
            pltpu.SemaphoreType.DMA((2,)),
        ],
    )(x, w1t, w2t)


# speedup vs baseline: 1.0768x; 1.0682x over previous
import jax
import jax.numpy as jnp
from jax.experimental import pallas as pl
from jax.experimental.pallas import tpu as pltpu

N_TOKENS = 16384
EMBED_DIM = 2048
LOW_RANK_DIM = 16
OUT_DIM = 64

BLOCK_T = 2048


def _fused_body(x_ref, w1t_ref, w2t_ref, out_ref):
    h = jnp.dot(x_ref[...], w1t_ref[...], preferred_element_type=jnp.float32)
    out_ref[...] = jnp.dot(h, w2t_ref[...], preferred_element_type=jnp.float32)


def kernel(x, w1, w2):
    n = x.shape[0]
    w1t = w1.T
    w2t = w2.T
    grid = (n // BLOCK_T,)
    return pl.pallas_call(
        _fused_body,
        grid=grid,
        in_specs=[
            pl.BlockSpec((BLOCK_T, EMBED_DIM), lambda i: (i, 0)),
            pl.BlockSpec((EMBED_DIM, LOW_RANK_DIM), lambda i: (0, 0)),
            pl.BlockSpec((LOW_RANK_DIM, OUT_DIM), lambda i: (0, 0)),
        ],
        out_specs=pl.BlockSpec((BLOCK_T, OUT_DIM), lambda i: (i, 0)),
        out_shape=jax.ShapeDtypeStruct((n, OUT_DIM), jnp.float32),
        compiler_params=pltpu.CompilerParams(
            dimension_semantics=("arbitrary",),
        ),
    )(x, w1t, w2t)
